# baseline (speedup 1.0000x reference)
import functools

import jax
import jax.numpy as jnp
from jax import lax
from jax.experimental import pallas as pl
from jax.experimental.pallas import tpu as pltpu
from jax.experimental.pallas import tpu_sc as plsc

N = 10000
D = 128
E = 320000
LANES = 16

NC = 2
NS = 16
NW = NC * NS
K = 128
EPAD = 10240
CHUNKS = EPAD // K
SUP = 16
NSUP = CHUNKS // SUP
SUPE = SUP * K
NP = 10240
RPT = NP // NS
KG = K // LANES


def _sc_agg(x, srcf, dst3, eaf):
    mesh = plsc.VectorSubcoreMesh(core_axis_name="c", subcore_axis_name="s")

    @functools.partial(
        pl.kernel,
        out_type=jax.ShapeDtypeStruct((NC, NS, RPT, D), jnp.float32),
        mesh=mesh,
        scratch_types=[
            pltpu.VMEM((SUP, K), jnp.int32),
            pltpu.VMEM((CHUNKS, K), jnp.int32),
            pltpu.VMEM((SUP, K), jnp.float32),
            [pltpu.VMEM((K, D), jnp.float32)] * 2,
            pltpu.VMEM_SHARED((NP, D), jnp.float32),
            [pltpu.SemaphoreType.DMA] * 2,
            [pltpu.SemaphoreType.DMA] * 2,
        ],
    )
    def k(x_hbm, src_hbm, dst_hbm, ea_hbm, out_hbm,
          src_v, dst_all, ea_v, rows, acc_sh, gsem, ssem):
        cid = lax.axis_index("c")
        sid = lax.axis_index("s")
        wid = cid * NS + sid

        def zrow(i, carry):
            for r in range(D // LANES):
                rows[0][i, pl.ds(r * LANES, LANES)] = jnp.zeros(
                    (LANES,), jnp.float32)
            return carry
        lax.fori_loop(0, K, zrow, 0)
        for t in range(RPT // K):
            pltpu.sync_copy(rows[0],
                            acc_sh.at[pl.ds(sid * RPT + t * K, K)])

        pltpu.sync_copy(dst_hbm.at[wid], dst_all)
        plsc.subcore_barrier()

        def gather_start(u, b):
            pltpu.async_copy(x_hbm.at[src_v.at[u]], rows[b], gsem[b])

        def gather_wait(b):
            pltpu.make_async_copy(x_hbm.at[src_v.at[0]],
                                  rows[b], gsem[b]).wait()

        def scatter_start(c, b):
            pltpu.async_copy(rows[b], acc_sh.at[dst_all.at[c]], ssem[b],
                             add=True)

        def scatter_wait(b):
            pltpu.make_async_copy(rows[b], acc_sh.at[dst_all.at[0]],
                                  ssem[b]).wait()

        def sup_body(si, carry):
            c0 = si * SUP
            pltpu.sync_copy(src_hbm.at[wid, pl.ds(c0, SUP)], src_v)
            pltpu.sync_copy(ea_hbm.at[wid, pl.ds(c0, SUP)], ea_v)

            @pl.when(si > 0)
            def _():
                scatter_wait(0)
            gather_start(0, 0)

            def pair(q, c2):
                for b in range(2):
                    u = 2 * q + b
                    c = c0 + u
                    gather_wait(b)
                    if b == 0:
                        @pl.when(jnp.logical_or(si > 0, q > 0))
                        def _():
                            scatter_wait(1)
                        gather_start(u + 1, 1)
                    else:
                        @pl.when(q < SUP // 2 - 1)
                        def _():
                            scatter_wait(0)
                            gather_start(u + 1, 0)

                    def scale(g, c3):
                        eav = ea_v[u, pl.ds(g * LANES, LANES)]
                        for li in range(LANES):
                            a = eav[li]
                            j = g * LANES + li
                            for r in range(D // LANES):
                                sl = pl.ds(r * LANES, LANES)
                                rows[b][j, sl] = rows[b][j, sl] * a
                        return c3
                    lax.fori_loop(0, KG, scale, 0)

                    scatter_start(c, b)
                return c2
            lax.fori_loop(0, SUP // 2, pair, 0)
            return carry
        lax.fori_loop(0, NSUP, sup_body, 0)

        scatter_wait(0)
        scatter_wait(1)
        plsc.subcore_barrier()

        pltpu.sync_copy(acc_sh.at[pl.ds(sid * RPT, RPT)],
                        out_hbm.at[cid, sid])

    return k(x, srcf, dst3, eaf).reshape(NC, NP, D)


_BN = 400


def _mm_body_relu(p_ref, w_ref, b_ref, o_ref):
    a = p_ref[0] + p_ref[1]
    h = jnp.dot(a, w_ref[...], preferred_element_type=jnp.float32)
    o_ref[...] = jnp.maximum(h + b_ref[...], 0.0)


def _mm_body_lin(p_ref, w_ref, b_ref, o_ref):
    a = p_ref[0] + p_ref[1]
    h = jnp.dot(a, w_ref[...], preferred_element_type=jnp.float32)
    o_ref[...] = h + b_ref[...]


def _mm(p, w, b, relu):
    body = _mm_body_relu if relu else _mm_body_lin
    return pl.pallas_call(
        body,
        grid=(N // _BN,),
        in_specs=[
            pl.BlockSpec((NC, _BN, D), lambda i: (0, i, 0)),
            pl.BlockSpec((D, D), lambda i: (0, 0)),
            pl.BlockSpec((1, D), lambda i: (0, 0)),
        ],
        out_specs=pl.BlockSpec((_BN, D), lambda i: (i, 0)),
        out_shape=jax.ShapeDtypeStruct((N, D), jnp.float32),
    )(p, w, b.reshape(1, D))


def kernel(x, edge_index, edge_attr, W1, b1, W2, b2):
    pad = NW * EPAD - E
    src = jnp.pad(edge_index[0], (0, pad)).reshape(NW, CHUNKS, K)
    dst = jnp.pad(edge_index[1], (0, pad)).reshape(NW, CHUNKS, K)
    ea = jnp.pad(edge_attr, (0, pad)).reshape(NW, CHUNKS, K)
    p1 = _sc_agg(x, src, dst, ea)
    h = _mm(p1, W1, b1, relu=True)
    p2 = _sc_agg(h, src, dst, ea)
    return _mm(p2, W2, b2, relu=False)
